# baseline (speedup 1.0000x reference)
import functools
import math

import jax
import jax.numpy as jnp
from jax import lax
from jax.experimental import pallas as pl
from jax.experimental.pallas import tpu as pltpu
from jax.experimental.pallas import tpu_sc as plsc

VOCAB = 1000000
EMB_DIM = 64
SCALE = math.sqrt(EMB_DIM)

_NC = 2
_NS = 16
_NW = _NC * _NS

_B_TOTAL = 4096 * 200
_B_PER_W = _B_TOTAL // _NW
_CHUNK = 128
_N_CHUNKS = _B_PER_W // _CHUNK
_SLICES_PER_ROW = EMB_DIM // 16


def _make_sc_gather():
    mesh = plsc.VectorSubcoreMesh(core_axis_name="c", subcore_axis_name="s")

    @functools.partial(
        pl.kernel,
        mesh=mesh,
        out_type=jax.ShapeDtypeStruct((_B_TOTAL, EMB_DIM), jnp.float32),
        compiler_params=pltpu.CompilerParams(use_tc_tiling_on_sc=False),
        scratch_types=[
            pltpu.VMEM((_CHUNK,), jnp.int32),
            pltpu.VMEM((_CHUNK, EMB_DIM), jnp.float32),
            pltpu.SemaphoreType.DMA,
        ],
    )
    def k(idx_hbm, table_hbm, out_hbm, idx_v, rows_v, sem):
        wid = lax.axis_index("s") * _NC + lax.axis_index("c")
        base = wid * _B_PER_W

        def chunk_body(g, carry):
            off = base + g * _CHUNK
            pltpu.sync_copy(idx_hbm.at[pl.ds(off, _CHUNK)], idx_v)
            pltpu.async_copy(table_hbm.at[idx_v], rows_v, sem).wait()

            def scale_body(r, c):
                for j in range(_SLICES_PER_ROW):
                    rows_v[r, pl.ds(j * 16, 16)] = (
                        rows_v[r, pl.ds(j * 16, 16)] * SCALE
                    )
                return c

            lax.fori_loop(0, _CHUNK, scale_body, 0)
            pltpu.sync_copy(rows_v, out_hbm.at[pl.ds(off, _CHUNK)])
            return carry

        lax.fori_loop(0, _N_CHUNKS, chunk_body, 0)

    return k


_sc_gather = _make_sc_gather()


def kernel(x, table):
    idx = x.reshape(-1).astype(jnp.int32)
    out = _sc_gather(idx, table)
    return out.reshape(x.shape[0], x.shape[1], EMB_DIM)


# speedup vs baseline: 1.2742x; 1.2742x over previous
import functools
import math

import jax
import jax.numpy as jnp
from jax import lax
from jax.experimental import pallas as pl
from jax.experimental.pallas import tpu as pltpu
from jax.experimental.pallas import tpu_sc as plsc

VOCAB = 1000000
EMB_DIM = 64
SCALE = math.sqrt(EMB_DIM)

_NC = 2
_NS = 16
_NW = _NC * _NS

_B_TOTAL = 4096 * 200
_B_PER_W = _B_TOTAL // _NW
_CHUNK = 128
_N_CHUNKS = _B_PER_W // _CHUNK
_NBUF = 8
_LOOKAHEAD = 4
_SLICES = EMB_DIM // 16


def _make_sc_gather():
    mesh = plsc.VectorSubcoreMesh(core_axis_name="c", subcore_axis_name="s")

    @functools.partial(
        pl.kernel,
        mesh=mesh,
        out_type=jax.ShapeDtypeStruct((_B_TOTAL, EMB_DIM), jnp.float32),
        compiler_params=pltpu.CompilerParams(use_tc_tiling_on_sc=False),
        scratch_types=[
            pltpu.VMEM((_N_CHUNKS, _CHUNK), jnp.int32),
            pltpu.VMEM((_NBUF, _CHUNK, EMB_DIM), jnp.float32),
        ]
        + [pltpu.SemaphoreType.DMA] * _NBUF
        + [pltpu.SemaphoreType.DMA] * _NBUF,
    )
    def k(idx_hbm, table_hbm, out_hbm, idx_v, rows_v, *sems):
        gsem = sems[:_NBUF]
        ssem = sems[_NBUF:]
        wid = lax.axis_index("s") * _NC + lax.axis_index("c")
        base = wid * _B_PER_W

        pltpu.sync_copy(idx_hbm.at[pl.ds(wid * _N_CHUNKS, _N_CHUNKS)], idx_v)

        def fire_gather(g, b):
            pltpu.async_copy(table_hbm.at[idx_v.at[g]], rows_v.at[b], gsem[b])

        def wait_gather(g, b):
            pltpu.make_async_copy(
                table_hbm.at[idx_v.at[g]], rows_v.at[b], gsem[b]
            ).wait()

        def fire_scatter(g, b):
            pltpu.async_copy(
                rows_v.at[b], out_hbm.at[pl.ds(base + g * _CHUNK, _CHUNK)],
                ssem[b],
            )

        def wait_scatter(g, b):
            pltpu.make_async_copy(
                rows_v.at[b], out_hbm.at[pl.ds(base + g * _CHUNK, _CHUNK)],
                ssem[b],
            ).wait()

        for b in range(_LOOKAHEAD):
            fire_gather(b, b)

        @pl.loop(0, _N_CHUNKS, step=_NBUF)
        def ring(G):
            for b in range(_NBUF):
                g = G + b
                wait_gather(g, b)

                @pl.loop(0, _CHUNK, unroll=4)
                def scale(r):
                    for j in range(_SLICES):
                        rows_v[b, r, pl.ds(j * 16, 16)] = (
                            rows_v[b, r, pl.ds(j * 16, 16)] * SCALE
                        )

                fire_scatter(g, b)

                bf = (b + _LOOKAHEAD) % _NBUF
                gf = g + _LOOKAHEAD

                @pl.when(gf < _N_CHUNKS)
                def _():
                    @pl.when(gf >= _NBUF)
                    def _():
                        wait_scatter(gf - _NBUF, bf)

                    fire_gather(gf, bf)

        for b in range(_NBUF):
            wait_scatter(_N_CHUNKS - _NBUF + b, b)

    return k


_sc_gather = _make_sc_gather()


def kernel(x, table):
    idx = x.reshape(_B_TOTAL // _CHUNK, _CHUNK).astype(jnp.int32)
    out = _sc_gather(idx, table)
    return out.reshape(x.shape[0], x.shape[1], EMB_DIM)
